# baseline (speedup 1.0000x reference)
import functools

import jax
import jax.numpy as jnp
from jax import lax
from jax.experimental import pallas as pl
from jax.experimental.pallas import tpu as pltpu
from jax.experimental.pallas import tpu_sc as plsc

_NB = 2000
_NC = 2
_NSUB = 16
_N = 100000
_ROWS = 100096
_CE = 512
_WB = 6256
_WB_LAST = _N - 15 * _WB


_B8 = 1600
_NP = 102400


def _cnt_expand(cnt8):
    lane = jax.lax.broadcasted_iota(jnp.int32, (8, 128), 1) // 16
    node = jax.lax.broadcasted_iota(jnp.int32, (8, 128), 0)
    k8 = (lane == node).astype(jnp.float32)
    return jnp.maximum(
        jnp.dot(cnt8, k8, preferred_element_type=jnp.float32), 1.0)


def _xform(lo, hi, bd_ref, bl_ref, bh_ref):
    t_lo = (jnp.dot(lo, bd_ref[0], preferred_element_type=jnp.float32)
            + jnp.dot(hi, bd_ref[1], preferred_element_type=jnp.float32)
            + bl_ref[...])
    t_hi = (jnp.dot(lo, bd_ref[2], preferred_element_type=jnp.float32)
            + jnp.dot(hi, bd_ref[3], preferred_element_type=jnp.float32)
            + bh_ref[...])
    return t_lo, t_hi


def _enc_body(xu_ref, xp_ref, xc_ref, bdu_ref, bul_ref, buh_ref,
              bdp_ref, bpl_ref, bph_ref, bdc_ref, bcl_ref, bch_ref,
              ceml_ref, cemh_ref, bd1p_ref, b1pl_ref, b1ph_ref,
              bd1c_ref, b1cl_ref, b1ch_ref, bd2c_ref, b2cl_ref, b2ch_ref,
              uhl_ref, uhh_ref, phl_ref, phh_ref, t1l_ref, t1h_ref,
              t3l_ref, t3h_ref, t7l_ref, t7h_ref):
    xu = xu_ref[...]
    uhl = jnp.maximum(jnp.dot(xu, bdu_ref[0],
                              preferred_element_type=jnp.float32)
                      + bul_ref[...], 0.0)
    uhh = jnp.maximum(jnp.dot(xu, bdu_ref[1],
                              preferred_element_type=jnp.float32)
                      + buh_ref[...], 0.0)
    xp = xp_ref[...]
    phl = jnp.maximum(jnp.dot(xp, bdp_ref[0],
                              preferred_element_type=jnp.float32)
                      + bpl_ref[...], 0.0)
    phh = jnp.maximum(jnp.dot(xp, bdp_ref[1],
                              preferred_element_type=jnp.float32)
                      + bph_ref[...], 0.0)
    uhl_ref[...] = uhl
    uhh_ref[...] = uhh
    phl_ref[...] = phl
    phh_ref[...] = phh
    t1l, t1h = _xform(phl, phh, bd1p_ref, b1pl_ref, b1ph_ref)
    t1l_ref[...] = t1l
    t1h_ref[...] = t1h

    @pl.when(pl.program_id(0) == 0)
    def _():
        xc = xc_ref[...]
        chl = jnp.maximum(jnp.dot(xc, bdc_ref[0],
                                  preferred_element_type=jnp.float32)
                          + bcl_ref[...], 0.0) + ceml_ref[...]
        chh = jnp.maximum(jnp.dot(xc, bdc_ref[1],
                                  preferred_element_type=jnp.float32)
                          + bch_ref[...], 0.0) + cemh_ref[...]
        t3l, t3h = _xform(chl, chh, bd1c_ref, b1cl_ref, b1ch_ref)
        t3l_ref[...] = t3l
        t3h_ref[...] = t3h
        t7l, t7h = _xform(chl, chh, bd2c_ref, b2cl_ref, b2ch_ref)
        t7l_ref[...] = t7l
        t7h_ref[...] = t7h


def _encoders(xu16, xp24, xc8, bdu, bu2, bdp, bp2, bdc, bc2, cem2,
              bd1p, b1p2, bd1c, b1c2, bd2c, b2c2):
    m = xu16.shape[0]
    mc = xc8.shape[0]
    grid = (m // _B8,)
    blk = lambda c: pl.BlockSpec((_B8, c), lambda i: (i, 0))
    full = lambda r, c: pl.BlockSpec((r, c), lambda i: (0, 0))
    pk = jax.ShapeDtypeStruct((m, 128), jnp.float32)
    pkc = jax.ShapeDtypeStruct((mc, 128), jnp.float32)
    return pl.pallas_call(
        _enc_body,
        grid=grid,
        in_specs=[blk(16), blk(24), full(mc, 8),
                  pl.BlockSpec((2, 16, 128), lambda i: (0, 0, 0)),
                  full(1, 128), full(1, 128),
                  pl.BlockSpec((2, 24, 128), lambda i: (0, 0, 0)),
                  full(1, 128), full(1, 128),
                  pl.BlockSpec((2, 8, 128), lambda i: (0, 0, 0)),
                  full(1, 128), full(1, 128),
                  full(mc, 128), full(mc, 128),
                  pl.BlockSpec((4, 128, 128), lambda i: (0, 0, 0)),
                  full(1, 128), full(1, 128),
                  pl.BlockSpec((4, 128, 128), lambda i: (0, 0, 0)),
                  full(1, 128), full(1, 128),
                  pl.BlockSpec((4, 128, 128), lambda i: (0, 0, 0)),
                  full(1, 128), full(1, 128)],
        out_specs=[blk(128), blk(128), blk(128), blk(128), blk(128),
                   blk(128), full(mc, 128), full(mc, 128), full(mc, 128),
                   full(mc, 128)],
        out_shape=[pk, pk, pk, pk, pk, pk, pkc, pkc, pkc, pkc],
    )(xu16, xp24, xc8, bdu, bu2[0], bu2[1], bdp, bp2[0], bp2[1],
      bdc, bc2[0], bc2[1], cem2[0], cem2[1], bd1p, b1p2[0], b1p2[1],
      bd1c, b1c2[0], b1c2[1], bd2c, b2c2[0], b2c2[1])


def _epi1_body(hl_ref, hh_ref, al_ref, ah_ref, cnt_ref, bd_ref, bl_ref,
               bh_ref, hlo_ref, hho_ref, tl_ref, th_ref):
    ce = _cnt_expand(cnt_ref[...])
    hl = hl_ref[...] + jnp.maximum(al_ref[...] / ce, 0.0)
    hh = hh_ref[...] + jnp.maximum(ah_ref[...] / ce, 0.0)
    hlo_ref[...] = hl
    hho_ref[...] = hh
    tl, th = _xform(hl, hh, bd_ref, bl_ref, bh_ref)
    tl_ref[...] = tl
    th_ref[...] = th


def _epi2_body(hl_ref, hh_ref, al_ref, ah_ref, cnt_a_ref,
               bl2_ref, bh2_ref, cnt_b_ref, bd_ref, bl_ref, bh_ref,
               hlo_ref, hho_ref, tl_ref, th_ref):
    ca = _cnt_expand(cnt_a_ref[...])
    cb = _cnt_expand(cnt_b_ref[...])
    g = jnp.where(pl.program_id(0) == 0, 1.0, 0.0)
    zpad = jnp.zeros((_B8 - 16, 128), jnp.float32)
    addl = jnp.concatenate([jnp.maximum(bl2_ref[...] / cb, 0.0), zpad], 0)
    addh = jnp.concatenate([jnp.maximum(bh2_ref[...] / cb, 0.0), zpad], 0)
    hl = hl_ref[...] + jnp.maximum(al_ref[...] / ca, 0.0) + g * addl
    hh = hh_ref[...] + jnp.maximum(ah_ref[...] / ca, 0.0) + g * addh
    hlo_ref[...] = hl
    hho_ref[...] = hh
    tl, th = _xform(hl, hh, bd_ref, bl_ref, bh_ref)
    tl_ref[...] = tl
    th_ref[...] = th


def _epiF_body(hl_ref, hh_ref, al_ref, ah_ref, cnt_ref, hlo_ref, hho_ref):
    ce = _cnt_expand(cnt_ref[...])
    hlo_ref[...] = hl_ref[...] + jnp.maximum(al_ref[...] / ce, 0.0)
    hho_ref[...] = hh_ref[...] + jnp.maximum(ah_ref[...] / ce, 0.0)


def _epi_single(hlh, agg_lh, cnt8, bd, b2):
    m = hlh[0].shape[0]
    grid = (m // _B8,)
    blk = lambda c: pl.BlockSpec((_B8, c), lambda i: (i, 0))
    full = lambda r, c: pl.BlockSpec((r, c), lambda i: (0, 0))
    pk = jax.ShapeDtypeStruct((m, 128), jnp.float32)
    return pl.pallas_call(
        _epi1_body, grid=grid,
        in_specs=[blk(128), blk(128), blk(128), blk(128), blk(8),
                  pl.BlockSpec((4, 128, 128), lambda i: (0, 0, 0)),
                  full(1, 128), full(1, 128)],
        out_specs=[blk(128), blk(128), blk(128), blk(128)],
        out_shape=[pk, pk, pk, pk],
    )(hlh[0], hlh[1], agg_lh[0], agg_lh[1], cnt8, bd, b2[0], b2[1])


def _epi_double(hlh, agg_a_lh, cnt_a8, agg_b_lh, cnt_b8, bd, b2):
    m = hlh[0].shape[0]
    grid = (m // _B8,)
    blk = lambda c: pl.BlockSpec((_B8, c), lambda i: (i, 0))
    full = lambda r, c: pl.BlockSpec((r, c), lambda i: (0, 0))
    pk = jax.ShapeDtypeStruct((m, 128), jnp.float32)
    return pl.pallas_call(
        _epi2_body, grid=grid,
        in_specs=[blk(128), blk(128), blk(128), blk(128), blk(8),
                  full(16, 128), full(16, 128), full(16, 8),
                  pl.BlockSpec((4, 128, 128), lambda i: (0, 0, 0)),
                  full(1, 128), full(1, 128)],
        out_specs=[blk(128), blk(128), blk(128), blk(128)],
        out_shape=[pk, pk, pk, pk],
    )(hlh[0], hlh[1], agg_a_lh[0], agg_a_lh[1], cnt_a8,
      agg_b_lh[0], agg_b_lh[1], cnt_b8, bd, b2[0], b2[1])


def _epi_final(hlh, agg_lh, cnt8):
    m = hlh[0].shape[0]
    grid = (m // _B8,)
    blk = lambda c: pl.BlockSpec((_B8, c), lambda i: (i, 0))
    pk = jax.ShapeDtypeStruct((m, 128), jnp.float32)
    return pl.pallas_call(
        _epiF_body, grid=grid,
        in_specs=[blk(128), blk(128), blk(128), blk(128), blk(8)],
        out_specs=[blk(128), blk(128)],
        out_shape=[pk, pk],
    )(hlh[0], hlh[1], agg_lh[0], agg_lh[1], cnt8)


def _sc_pass(t_lo, t_hi, src, dst, ea, *, with_cnt):
    weighted = ea is not None
    e_pad = src.shape[0]
    n_my = e_pad // (_CE * _NSUB)

    def body(*refs):
        iota16 = lax.iota(jnp.int32, 16)
        it = iter(refs)
        t_lo_h = next(it)
        t_hi_h = next(it)
        src_h = next(it)
        dst_h = next(it)
        ea_h = next(it) if weighted else None
        out_lo = next(it)
        out_hi = next(it)
        cnt_out = next(it) if with_cnt else None
        agg_sh = next(it)
        cnt_sh = next(it) if with_cnt else None
        src_v = (next(it), next(it), next(it))
        dst_v = (next(it), next(it))
        ea_v = (next(it), next(it), next(it)) if weighted else None
        rows_v = (next(it), next(it))
        wrk_v = next(it) if with_cnt else None
        sem_l = (next(it), next(it), next(it))
        sem_g = (next(it), next(it))
        sem_s = (next(it), next(it))
        sem_c = (next(it), next(it)) if with_cnt else None

        cid = lax.axis_index("c")
        sid = lax.axis_index("s")

        def zrow(i, _):
            rows_v[0][i, :] = jnp.zeros((16,), jnp.float32)
            return 0
        lax.fori_loop(0, _CE, zrow, 0)
        zb = sid * _WB
        zfull, ztail = _WB // _CE, _WB % _CE
        for q in range(zfull):
            pltpu.sync_copy(rows_v[0], agg_sh.at[pl.ds(zb + q * _CE, _CE)])
        pltpu.sync_copy(rows_v[0].at[pl.ds(0, ztail)],
                        agg_sh.at[pl.ds(zb + zfull * _CE, ztail)])

        if with_cnt:
            @pl.when(cid == 0)
            def _():
                def zw(i, _):
                    wrk_v[pl.ds(i * 16, 16)] = jnp.zeros((16,), jnp.float32)
                    return 0
                lax.fori_loop(0, _CE // 16, zw, 0)
                for q in range(_WB // _CE):
                    pltpu.sync_copy(wrk_v, cnt_sh.at[pl.ds(zb + q * _CE, _CE)])
                pltpu.sync_copy(wrk_v.at[pl.ds(0, _WB % _CE)],
                                cnt_sh.at[pl.ds(zb + (_WB // _CE) * _CE,
                                                _WB % _CE)])

                def ow(i, _):
                    wrk_v[pl.ds(i * 16, 16)] = jnp.ones((16,), jnp.float32)
                    return 0
                lax.fori_loop(0, _CE // 16, ow, 0)

        plsc.subcore_barrier()

        base = sid * n_my
        ng = n_my // 6

        def ld_issue(i, b3):
            cb = (base + i) * _CE
            pltpu.async_copy(src_h.at[pl.ds(cb, _CE)], src_v[b3], sem_l[b3])
            if weighted:
                pltpu.async_copy(ea_h.at[pl.ds(cb, _CE)], ea_v[b3], sem_l[b3])

        def ld_wait(b3):
            pltpu.make_async_copy(src_h.at[pl.ds(0, _CE)], src_v[b3],
                                  sem_l[b3]).wait()
            if weighted:
                pltpu.make_async_copy(ea_h.at[pl.ds(0, _CE)], ea_v[b3],
                                      sem_l[b3]).wait()

        def gather_issue(b2, b3):
            @pl.when(cid == 0)
            def _():
                pltpu.async_copy(t_lo_h.at[src_v[b3]], rows_v[b2], sem_g[b2])

            @pl.when(cid == 1)
            def _():
                pltpu.async_copy(t_hi_h.at[src_v[b3]], rows_v[b2], sem_g[b2])

        def gather_wait(b2, b3):
            pltpu.make_async_copy(t_lo_h.at[src_v[b3]], rows_v[b2],
                                  sem_g[b2]).wait()

        def scatter_issue(b2):
            pltpu.async_copy(rows_v[b2], agg_sh.at[dst_v[b2]], sem_s[b2],
                             add=True)
            if with_cnt:
                @pl.when(cid == 0)
                def _():
                    pltpu.async_copy(wrk_v, cnt_sh.at[dst_v[b2]], sem_c[b2],
                                     add=True)

        def scatter_wait(b2):
            pltpu.make_async_copy(rows_v[b2], agg_sh.at[dst_v[b2]],
                                  sem_s[b2]).wait()
            if with_cnt:
                @pl.when(cid == 0)
                def _():
                    pltpu.make_async_copy(wrk_v, cnt_sh.at[dst_v[b2]],
                                          sem_c[b2]).wait()

        def scale_chunk(b2, b3):
            if weighted:
                def scale(g, _):
                    eav = ea_v[b3][pl.ds(g * 16, 16)]
                    for u in range(16):
                        e = g * 16 + u
                        rows_v[b2][e, :] = rows_v[b2][e, :] * eav[u]
                    return 0
                lax.fori_loop(0, _CE // 16, scale, 0)

        ld_issue(0, 0)

        def group(k, _):
            for u in range(6):
                b2, b3 = u % 2, u % 3
                ld_wait(b3)
                if u >= 2:
                    scatter_wait(b2)
                else:
                    @pl.when(k > 0)
                    def _():
                        scatter_wait(b2)
                gather_issue(b2, b3)
                i = k * 6 + u
                if u < 5:
                    ld_issue(i + 1, (u + 1) % 3)
                else:
                    @pl.when(k < ng - 1)
                    def _():
                        ld_issue(i + 1, 0)
                cb = (base + i) * _CE
                pltpu.sync_copy(dst_h.at[pl.ds(cb, _CE)], dst_v[b2])
                pb2, pb3 = (u - 1) % 2, (u - 1) % 3
                if u >= 1:
                    gather_wait(pb2, pb3)
                    scale_chunk(pb2, pb3)
                    scatter_issue(pb2)
                else:
                    @pl.when(k > 0)
                    def _():
                        gather_wait(pb2, pb3)
                        scale_chunk(pb2, pb3)
                        scatter_issue(pb2)
            return 0
        lax.fori_loop(0, ng, group, 0)
        gather_wait(1, 2)
        scale_chunk(1, 2)
        scatter_issue(1)
        scatter_wait(0)
        scatter_wait(1)

        plsc.subcore_barrier()

        wb = sid * _WB

        @pl.when(jnp.logical_and(cid == 0, sid < _NSUB - 1))
        def _():
            pltpu.sync_copy(agg_sh.at[pl.ds(wb, _WB)], out_lo.at[pl.ds(wb, _WB)])

        @pl.when(jnp.logical_and(cid == 0, sid == _NSUB - 1))
        def _():
            pltpu.sync_copy(agg_sh.at[pl.ds(15 * _WB, _WB_LAST)],
                            out_lo.at[pl.ds(15 * _WB, _WB_LAST)])

        @pl.when(jnp.logical_and(cid == 1, sid < _NSUB - 1))
        def _():
            pltpu.sync_copy(agg_sh.at[pl.ds(wb, _WB)], out_hi.at[pl.ds(wb, _WB)])

        @pl.when(jnp.logical_and(cid == 1, sid == _NSUB - 1))
        def _():
            pltpu.sync_copy(agg_sh.at[pl.ds(15 * _WB, _WB_LAST)],
                            out_hi.at[pl.ds(15 * _WB, _WB_LAST)])

        if with_cnt:
            @pl.when(jnp.logical_and(cid == 0, sid < _NSUB - 1))
            def _():
                pltpu.sync_copy(cnt_sh.at[pl.ds(wb, _WB)],
                                cnt_out.at[pl.ds(wb, _WB)])

            @pl.when(jnp.logical_and(cid == 0, sid == _NSUB - 1))
            def _():
                pltpu.sync_copy(cnt_sh.at[pl.ds(15 * _WB, _WB_LAST)],
                                cnt_out.at[pl.ds(15 * _WB, _WB_LAST)])

    out_type = [jax.ShapeDtypeStruct((_NP, 16), jnp.float32),
                jax.ShapeDtypeStruct((_NP, 16), jnp.float32)]
    if with_cnt:
        out_type.append(jax.ShapeDtypeStruct((_NP,), jnp.float32))
    scratch = [pltpu.VMEM_SHARED((_ROWS, 16), jnp.float32)]
    if with_cnt:
        scratch.append(pltpu.VMEM_SHARED((_ROWS,), jnp.float32))
    scratch += [pltpu.VMEM((_CE,), jnp.int32)] * 3
    scratch += [pltpu.VMEM((_CE,), jnp.int32)] * 2
    if weighted:
        scratch += [pltpu.VMEM((_CE,), jnp.float32)] * 3
    scratch += [pltpu.VMEM((_CE, 16), jnp.float32)] * 2
    if with_cnt:
        scratch.append(pltpu.VMEM((_CE,), jnp.float32))
    scratch += [pltpu.SemaphoreType.DMA] * 7
    if with_cnt:
        scratch += [pltpu.SemaphoreType.DMA] * 2

    mesh = plsc.VectorSubcoreMesh(core_axis_name="c", subcore_axis_name="s",
                                  num_cores=_NC, num_subcores=_NSUB)
    args = [t_lo, t_hi, src, dst] + ([ea] if weighted else [])
    return pl.kernel(body, out_type=out_type, mesh=mesh,
                     compiler_params=pltpu.CompilerParams(
                         use_tc_tiling_on_sc=False),
                     scratch_types=scratch)(*args)


def _sc_belongs(t3l, t3h, t7l, t7h, src, dst):
    e_pad = src.shape[0]
    n_my = e_pad // (_CE * _NSUB)

    def body(t3l_h, t3h_h, t7l_h, t7h_h, src_h, dst_h,
             o3l, o3h, o7l, o7h, cnt_out,
             a3_sh, a7_sh, cnt_sh, src_v, dst_v, rows_v, wrk_v, sem_g):
        cid = lax.axis_index("c")
        sid = lax.axis_index("s")

        def zrow(i, _):
            rows_v[i, :] = jnp.zeros((16,), jnp.float32)
            return 0
        lax.fori_loop(0, 8, zrow, 0)

        def zw(i, _):
            wrk_v[pl.ds(i * 16, 16)] = jnp.zeros((16,), jnp.float32)
            return 0
        lax.fori_loop(0, _CE // 16, zw, 0)

        zb = sid * 8
        pltpu.sync_copy(rows_v.at[pl.ds(0, 8)], a3_sh.at[pl.ds(zb, 8)])
        pltpu.sync_copy(rows_v.at[pl.ds(0, 8)], a7_sh.at[pl.ds(zb, 8)])

        @pl.when(cid == 0)
        def _():
            pltpu.sync_copy(wrk_v.at[pl.ds(0, 8)], cnt_sh.at[pl.ds(zb, 8)])

            def ow(i, _):
                wrk_v[pl.ds(i * 16, 16)] = jnp.ones((16,), jnp.float32)
                return 0
            lax.fori_loop(0, _CE // 16, ow, 0)

        plsc.subcore_barrier()

        def chunk(j, _):
            cb = (sid * n_my + j) * _CE
            pltpu.sync_copy(src_h.at[pl.ds(cb, _CE)], src_v)
            pltpu.sync_copy(dst_h.at[pl.ds(cb, _CE)], dst_v)

            @pl.when(cid == 0)
            def _():
                pltpu.async_copy(t3l_h.at[src_v], rows_v, sem_g).wait()
                pltpu.sync_copy(rows_v, a3_sh.at[dst_v], add=True)
                pltpu.async_copy(t7l_h.at[src_v], rows_v, sem_g).wait()
                pltpu.sync_copy(rows_v, a7_sh.at[dst_v], add=True)
                pltpu.sync_copy(wrk_v, cnt_sh.at[dst_v], add=True)

            @pl.when(cid == 1)
            def _():
                pltpu.async_copy(t3h_h.at[src_v], rows_v, sem_g).wait()
                pltpu.sync_copy(rows_v, a3_sh.at[dst_v], add=True)
                pltpu.async_copy(t7h_h.at[src_v], rows_v, sem_g).wait()
                pltpu.sync_copy(rows_v, a7_sh.at[dst_v], add=True)
            return 0
        lax.fori_loop(0, n_my, chunk, 0)

        plsc.subcore_barrier()

        def zrow2(i, _):
            rows_v[i, :] = jnp.zeros((16,), jnp.float32)
            return 0
        lax.fori_loop(0, 8, zrow2, 0)
        wb = sid * 8

        @pl.when(jnp.logical_and(cid == 0, sid < 15))
        def _():
            pltpu.sync_copy(a3_sh.at[pl.ds(wb, 8)], o3l.at[pl.ds(wb, 8)])
            pltpu.sync_copy(a7_sh.at[pl.ds(wb, 8)], o7l.at[pl.ds(wb, 8)])
            pltpu.sync_copy(cnt_sh.at[pl.ds(wb, 8)], cnt_out.at[pl.ds(wb, 8)])

        @pl.when(jnp.logical_and(cid == 1, sid < 15))
        def _():
            pltpu.sync_copy(a3_sh.at[pl.ds(wb, 8)], o3h.at[pl.ds(wb, 8)])
            pltpu.sync_copy(a7_sh.at[pl.ds(wb, 8)], o7h.at[pl.ds(wb, 8)])

        @pl.when(jnp.logical_and(cid == 0, sid == 15))
        def _():
            pltpu.sync_copy(rows_v.at[pl.ds(0, 8)], o3l.at[pl.ds(120, 8)])
            pltpu.sync_copy(rows_v.at[pl.ds(0, 8)], o7l.at[pl.ds(120, 8)])

            def zw2(i, _):
                wrk_v[pl.ds(i * 16, 16)] = jnp.zeros((16,), jnp.float32)
                return 0
            lax.fori_loop(0, 1, zw2, 0)
            pltpu.sync_copy(wrk_v.at[pl.ds(0, 8)], cnt_out.at[pl.ds(120, 8)])

        @pl.when(jnp.logical_and(cid == 1, sid == 15))
        def _():
            pltpu.sync_copy(rows_v.at[pl.ds(0, 8)], o3h.at[pl.ds(120, 8)])
            pltpu.sync_copy(rows_v.at[pl.ds(0, 8)], o7h.at[pl.ds(120, 8)])

    sm = jax.ShapeDtypeStruct((128, 16), jnp.float32)
    out_type = [sm, sm, sm, sm, jax.ShapeDtypeStruct((128,), jnp.float32)]
    scratch = [pltpu.VMEM_SHARED((128, 16), jnp.float32),
               pltpu.VMEM_SHARED((128, 16), jnp.float32),
               pltpu.VMEM_SHARED((128,), jnp.float32),
               pltpu.VMEM((_CE,), jnp.int32), pltpu.VMEM((_CE,), jnp.int32),
               pltpu.VMEM((_CE, 16), jnp.float32),
               pltpu.VMEM((_CE,), jnp.float32),
               pltpu.SemaphoreType.DMA]
    mesh = plsc.VectorSubcoreMesh(core_axis_name="c", subcore_axis_name="s",
                                  num_cores=_NC, num_subcores=_NSUB)
    return pl.kernel(body, out_type=out_type, mesh=mesh,
                     compiler_params=pltpu.CompilerParams(
                         use_tc_tiling_on_sc=False),
                     scratch_types=scratch)(t3l, t3h, t7l, t7h, src, dst)


def _pad_edges(ei, ea, dump=_N, gran=6 * _CE * _NSUB):
    src, dst = ei[0], ei[1]
    e = src.shape[0]
    e_pad = -(-e // gran) * gran
    pad = e_pad - e
    src = jnp.concatenate([src, jnp.zeros((pad,), src.dtype)])
    dst = jnp.concatenate([dst, jnp.full((pad,), dump, dst.dtype)])
    if ea is not None:
        ea = jnp.concatenate([ea, jnp.zeros((pad,), ea.dtype)])
    return src, dst, ea


def kernel(x_user, x_poi, x_cate, ea_pv, ea_rev_pv, ea_trans,
           user_lin_W, user_lin_b, poi_lin_W, poi_lin_b, cate_lin_W,
           cate_lin_b, cate_emb,
           W1_u_W, W1_u_b, W1_p_W, W1_p_b, W1_c_W, W1_c_b, W1_pp_W, W1_pp_b,
           W2_u_W, W2_u_b, W2_p_W, W2_p_b, W2_c_W, W2_c_b, W2_pp_W, W2_pp_b,
           ei_pv, ei_rev_pv, ei_belongs, ei_trans):
    nc = x_cate.shape[0]

    s_rev, d_rev, ea_rev = _pad_edges(ei_rev_pv, ea_rev_pv)
    s_pv, d_pv, ea_pvp = _pad_edges(ei_pv, ea_pv)
    s_bl, d_bl, _ = _pad_edges(ei_belongs, None, dump=120, gran=_CE * _NSUB)
    s_tr, d_tr, ea_trp = _pad_edges(ei_trans, ea_trans)

    i8 = jnp.eye(8, dtype=jnp.float32)

    def bd4(w):
        wt = w.T
        return jnp.stack([jnp.kron(i8, wt[:16, :16]),
                          jnp.kron(i8, wt[16:, :16]),
                          jnp.kron(i8, wt[:16, 16:]),
                          jnp.kron(i8, wt[16:, 16:])])

    def bd2(w):
        wt = w.T
        return jnp.stack([jnp.kron(i8, wt[:, :16]), jnp.kron(i8, wt[:, 16:])])

    def pb(b):
        return jnp.stack([jnp.tile(b[:16], 8), jnp.tile(b[16:], 8)])[:, None, :]

    xu16 = jnp.concatenate([x_user.reshape(-1, 16),
                            jnp.zeros((300, 16), jnp.float32)])
    xp24 = jnp.concatenate([x_poi.reshape(-1, 24),
                            jnp.zeros((300, 24), jnp.float32)])
    xc8 = jnp.concatenate([x_cate,
                           jnp.zeros((4, 1), jnp.float32)]).reshape(-1, 8)
    cem = jnp.concatenate([cate_emb[:nc], jnp.zeros((4, 32), jnp.float32)])
    cem2 = jnp.stack([cem[:, :16].reshape(-1, 128),
                      cem[:, 16:].reshape(-1, 128)])

    (uhl, uhh, phl, phh, t1l, t1h, t3l, t3h, t7l, t7h) = _encoders(
        xu16, xp24, xc8, bd2(user_lin_W), pb(user_lin_b),
        bd2(poi_lin_W), pb(poi_lin_b), bd2(cate_lin_W), pb(cate_lin_b),
        cem2, bd4(W1_p_W), pb(W1_p_b), bd4(W1_c_W), pb(W1_c_b),
        bd4(W2_c_W), pb(W2_c_b))

    def sc(tl, th, s, d, ea, with_cnt):
        r = _sc_pass(tl.reshape(-1, 16), th.reshape(-1, 16), s, d, ea,
                     with_cnt=with_cnt)
        if with_cnt:
            return (r[0].reshape(-1, 128), r[1].reshape(-1, 128),
                    r[2].reshape(-1, 8))
        return r[0].reshape(-1, 128), r[1].reshape(-1, 128)

    a3l, a3h, a7l, a7h, cnt_bl = _sc_belongs(
        t3l.reshape(-1, 16), t3h.reshape(-1, 16),
        t7l.reshape(-1, 16), t7h.reshape(-1, 16), s_bl, d_bl)
    a3lh = (a3l.reshape(-1, 128), a3h.reshape(-1, 128))
    a7lh = (a7l.reshape(-1, 128), a7h.reshape(-1, 128))
    cnt_bl8 = cnt_bl.reshape(-1, 8)

    a1l, a1h, cnt_rev = sc(t1l, t1h, s_rev, d_rev, ea_rev, True)
    uhl1, uhh1, t2l, t2h = _epi_single((uhl, uhh), (a1l, a1h), cnt_rev,
                                       bd4(W1_u_W), pb(W1_u_b))

    a2l, a2h, cnt_pv = sc(t2l, t2h, s_pv, d_pv, ea_pvp, True)
    phl1a, phh1a, t4l, t4h = _epi_double((phl, phh), (a2l, a2h), cnt_pv,
                                         a3lh, cnt_bl8,
                                         bd4(W1_pp_W), pb(W1_pp_b))

    a4l, a4h, cnt_tr = sc(t4l, t4h, s_tr, d_tr, ea_trp, True)
    phl1, phh1, t5l, t5h = _epi_single((phl1a, phh1a), (a4l, a4h), cnt_tr,
                                       bd4(W2_p_W), pb(W2_p_b))

    a5l, a5h = sc(t5l, t5h, s_rev, d_rev, ea_rev, False)
    uhl2, uhh2, t6l, t6h = _epi_single((uhl1, uhh1), (a5l, a5h), cnt_rev,
                                       bd4(W2_u_W), pb(W2_u_b))

    a6l, a6h = sc(t6l, t6h, s_pv, d_pv, ea_pvp, False)
    phl2a, phh2a, t8l, t8h = _epi_double((phl1, phh1), (a6l, a6h), cnt_pv,
                                         a7lh, cnt_bl8,
                                         bd4(W2_pp_W), pb(W2_pp_b))

    a8l, a8h = sc(t8l, t8h, s_tr, d_tr, ea_trp, False)
    pol, poh = _epi_final((phl2a, phh2a), (a8l, a8h), cnt_tr)
    return jnp.concatenate([pol.reshape(-1, 16)[: x_poi.shape[0]],
                            poh.reshape(-1, 16)[: x_poi.shape[0]]], axis=1)


# speedup vs baseline: 1.0597x; 1.0597x over previous
import functools

import jax
import jax.numpy as jnp
from jax import lax
from jax.experimental import pallas as pl
from jax.experimental.pallas import tpu as pltpu
from jax.experimental.pallas import tpu_sc as plsc

_NB = 2000
_NC = 2
_NSUB = 16
_N = 100000
_ROWS = 100096
_CE = 512
_WB = 6256
_WB_LAST = _N - 15 * _WB


_B8 = 1600
_NP = 102400


def _cnt_expand(cnt8):
    lane = jax.lax.broadcasted_iota(jnp.int32, (8, 128), 1) // 16
    node = jax.lax.broadcasted_iota(jnp.int32, (8, 128), 0)
    k8 = (lane == node).astype(jnp.float32)
    return jnp.maximum(
        jnp.dot(cnt8, k8, preferred_element_type=jnp.float32), 1.0)


def _xform(lo, hi, bd_ref, bl_ref, bh_ref):
    t_lo = (jnp.dot(lo, bd_ref[0], preferred_element_type=jnp.float32)
            + jnp.dot(hi, bd_ref[1], preferred_element_type=jnp.float32)
            + bl_ref[...])
    t_hi = (jnp.dot(lo, bd_ref[2], preferred_element_type=jnp.float32)
            + jnp.dot(hi, bd_ref[3], preferred_element_type=jnp.float32)
            + bh_ref[...])
    return t_lo, t_hi


def _enc_body(xu_ref, xp_ref, xc_ref, bdu_ref, bul_ref, buh_ref,
              bdp_ref, bpl_ref, bph_ref, bdc_ref, bcl_ref, bch_ref,
              ceml_ref, cemh_ref, bd1p_ref, b1pl_ref, b1ph_ref,
              bd1c_ref, b1cl_ref, b1ch_ref, bd2c_ref, b2cl_ref, b2ch_ref,
              uhl_ref, uhh_ref, phl_ref, phh_ref, t1l_ref, t1h_ref,
              t3l_ref, t3h_ref, t7l_ref, t7h_ref):
    xu = xu_ref[...]
    uhl = jnp.maximum(jnp.dot(xu, bdu_ref[0],
                              preferred_element_type=jnp.float32)
                      + bul_ref[...], 0.0)
    uhh = jnp.maximum(jnp.dot(xu, bdu_ref[1],
                              preferred_element_type=jnp.float32)
                      + buh_ref[...], 0.0)
    xp = xp_ref[...]
    phl = jnp.maximum(jnp.dot(xp, bdp_ref[0],
                              preferred_element_type=jnp.float32)
                      + bpl_ref[...], 0.0)
    phh = jnp.maximum(jnp.dot(xp, bdp_ref[1],
                              preferred_element_type=jnp.float32)
                      + bph_ref[...], 0.0)
    uhl_ref[...] = uhl
    uhh_ref[...] = uhh
    phl_ref[...] = phl
    phh_ref[...] = phh
    t1l, t1h = _xform(phl, phh, bd1p_ref, b1pl_ref, b1ph_ref)
    t1l_ref[...] = t1l
    t1h_ref[...] = t1h

    @pl.when(pl.program_id(0) == 0)
    def _():
        xc = xc_ref[...]
        chl = jnp.maximum(jnp.dot(xc, bdc_ref[0],
                                  preferred_element_type=jnp.float32)
                          + bcl_ref[...], 0.0) + ceml_ref[...]
        chh = jnp.maximum(jnp.dot(xc, bdc_ref[1],
                                  preferred_element_type=jnp.float32)
                          + bch_ref[...], 0.0) + cemh_ref[...]
        t3l, t3h = _xform(chl, chh, bd1c_ref, b1cl_ref, b1ch_ref)
        t3l_ref[...] = t3l
        t3h_ref[...] = t3h
        t7l, t7h = _xform(chl, chh, bd2c_ref, b2cl_ref, b2ch_ref)
        t7l_ref[...] = t7l
        t7h_ref[...] = t7h


def _encoders(xu16, xp24, xc8, bdu, bu2, bdp, bp2, bdc, bc2, cem2,
              bd1p, b1p2, bd1c, b1c2, bd2c, b2c2):
    m = xu16.shape[0]
    mc = xc8.shape[0]
    grid = (m // _B8,)
    blk = lambda c: pl.BlockSpec((_B8, c), lambda i: (i, 0))
    full = lambda r, c: pl.BlockSpec((r, c), lambda i: (0, 0))
    pk = jax.ShapeDtypeStruct((m, 128), jnp.float32)
    pkc = jax.ShapeDtypeStruct((mc, 128), jnp.float32)
    return pl.pallas_call(
        _enc_body,
        grid=grid,
        in_specs=[blk(16), blk(24), full(mc, 8),
                  pl.BlockSpec((2, 16, 128), lambda i: (0, 0, 0)),
                  full(1, 128), full(1, 128),
                  pl.BlockSpec((2, 24, 128), lambda i: (0, 0, 0)),
                  full(1, 128), full(1, 128),
                  pl.BlockSpec((2, 8, 128), lambda i: (0, 0, 0)),
                  full(1, 128), full(1, 128),
                  full(mc, 128), full(mc, 128),
                  pl.BlockSpec((4, 128, 128), lambda i: (0, 0, 0)),
                  full(1, 128), full(1, 128),
                  pl.BlockSpec((4, 128, 128), lambda i: (0, 0, 0)),
                  full(1, 128), full(1, 128),
                  pl.BlockSpec((4, 128, 128), lambda i: (0, 0, 0)),
                  full(1, 128), full(1, 128)],
        out_specs=[blk(128), blk(128), blk(128), blk(128), blk(128),
                   blk(128), full(mc, 128), full(mc, 128), full(mc, 128),
                   full(mc, 128)],
        out_shape=[pk, pk, pk, pk, pk, pk, pkc, pkc, pkc, pkc],
    )(xu16, xp24, xc8, bdu, bu2[0], bu2[1], bdp, bp2[0], bp2[1],
      bdc, bc2[0], bc2[1], cem2[0], cem2[1], bd1p, b1p2[0], b1p2[1],
      bd1c, b1c2[0], b1c2[1], bd2c, b2c2[0], b2c2[1])


def _epi1_body(hl_ref, hh_ref, al_ref, ah_ref, cnt_ref, bd_ref, bl_ref,
               bh_ref, hlo_ref, hho_ref, tl_ref, th_ref):
    ce = _cnt_expand(cnt_ref[...])
    hl = hl_ref[...] + jnp.maximum(al_ref[...] / ce, 0.0)
    hh = hh_ref[...] + jnp.maximum(ah_ref[...] / ce, 0.0)
    hlo_ref[...] = hl
    hho_ref[...] = hh
    tl, th = _xform(hl, hh, bd_ref, bl_ref, bh_ref)
    tl_ref[...] = tl
    th_ref[...] = th


def _epi2_body(hl_ref, hh_ref, al_ref, ah_ref, cnt_a_ref,
               bl2_ref, bh2_ref, cnt_b_ref, bd_ref, bl_ref, bh_ref,
               hlo_ref, hho_ref, tl_ref, th_ref):
    ca = _cnt_expand(cnt_a_ref[...])
    cb = _cnt_expand(cnt_b_ref[...])
    g = jnp.where(pl.program_id(0) == 0, 1.0, 0.0)
    zpad = jnp.zeros((_B8 - 16, 128), jnp.float32)
    addl = jnp.concatenate([jnp.maximum(bl2_ref[...] / cb, 0.0), zpad], 0)
    addh = jnp.concatenate([jnp.maximum(bh2_ref[...] / cb, 0.0), zpad], 0)
    hl = hl_ref[...] + jnp.maximum(al_ref[...] / ca, 0.0) + g * addl
    hh = hh_ref[...] + jnp.maximum(ah_ref[...] / ca, 0.0) + g * addh
    hlo_ref[...] = hl
    hho_ref[...] = hh
    tl, th = _xform(hl, hh, bd_ref, bl_ref, bh_ref)
    tl_ref[...] = tl
    th_ref[...] = th


def _epiF_body(hl_ref, hh_ref, al_ref, ah_ref, cnt_ref, hlo_ref, hho_ref):
    ce = _cnt_expand(cnt_ref[...])
    hlo_ref[...] = hl_ref[...] + jnp.maximum(al_ref[...] / ce, 0.0)
    hho_ref[...] = hh_ref[...] + jnp.maximum(ah_ref[...] / ce, 0.0)


def _epi_single(hlh, agg_lh, cnt8, bd, b2):
    m = hlh[0].shape[0]
    grid = (m // _B8,)
    blk = lambda c: pl.BlockSpec((_B8, c), lambda i: (i, 0))
    full = lambda r, c: pl.BlockSpec((r, c), lambda i: (0, 0))
    pk = jax.ShapeDtypeStruct((m, 128), jnp.float32)
    return pl.pallas_call(
        _epi1_body, grid=grid,
        in_specs=[blk(128), blk(128), blk(128), blk(128), blk(8),
                  pl.BlockSpec((4, 128, 128), lambda i: (0, 0, 0)),
                  full(1, 128), full(1, 128)],
        out_specs=[blk(128), blk(128), blk(128), blk(128)],
        out_shape=[pk, pk, pk, pk],
    )(hlh[0], hlh[1], agg_lh[0], agg_lh[1], cnt8, bd, b2[0], b2[1])


def _epi_double(hlh, agg_a_lh, cnt_a8, agg_b_lh, cnt_b8, bd, b2):
    m = hlh[0].shape[0]
    grid = (m // _B8,)
    blk = lambda c: pl.BlockSpec((_B8, c), lambda i: (i, 0))
    full = lambda r, c: pl.BlockSpec((r, c), lambda i: (0, 0))
    pk = jax.ShapeDtypeStruct((m, 128), jnp.float32)
    return pl.pallas_call(
        _epi2_body, grid=grid,
        in_specs=[blk(128), blk(128), blk(128), blk(128), blk(8),
                  full(16, 128), full(16, 128), full(16, 8),
                  pl.BlockSpec((4, 128, 128), lambda i: (0, 0, 0)),
                  full(1, 128), full(1, 128)],
        out_specs=[blk(128), blk(128), blk(128), blk(128)],
        out_shape=[pk, pk, pk, pk],
    )(hlh[0], hlh[1], agg_a_lh[0], agg_a_lh[1], cnt_a8,
      agg_b_lh[0], agg_b_lh[1], cnt_b8, bd, b2[0], b2[1])


def _epi_final(hlh, agg_lh, cnt8):
    m = hlh[0].shape[0]
    grid = (m // _B8,)
    blk = lambda c: pl.BlockSpec((_B8, c), lambda i: (i, 0))
    pk = jax.ShapeDtypeStruct((m, 128), jnp.float32)
    return pl.pallas_call(
        _epiF_body, grid=grid,
        in_specs=[blk(128), blk(128), blk(128), blk(128), blk(8)],
        out_specs=[blk(128), blk(128)],
        out_shape=[pk, pk],
    )(hlh[0], hlh[1], agg_lh[0], agg_lh[1], cnt8)


def _sc_pass(t_lo, t_hi, src, dst, ea, *, with_cnt):
    weighted = ea is not None
    e_pad = src.shape[0]
    n_my = e_pad // (_CE * _NSUB)

    def body(*refs):
        iota16 = lax.iota(jnp.int32, 16)
        it = iter(refs)
        t_lo_h = next(it)
        t_hi_h = next(it)
        src_h = next(it)
        dst_h = next(it)
        ea_h = next(it) if weighted else None
        out_lo = next(it)
        out_hi = next(it)
        cnt_out = next(it) if with_cnt else None
        agg_sh = next(it)
        cnt_sh = next(it) if with_cnt else None
        src_v = (next(it), next(it))
        dst_v = (next(it), next(it))
        ea_v = (next(it), next(it)) if weighted else None
        rows_v = (next(it), next(it))
        wrk_v = next(it) if with_cnt else None
        sem_l = (next(it), next(it))
        sem_g = (next(it), next(it))
        sem_s = (next(it), next(it))
        sem_c = (next(it), next(it)) if with_cnt else None

        cid = lax.axis_index("c")
        sid = lax.axis_index("s")

        def zrow(i, _):
            rows_v[0][i, :] = jnp.zeros((16,), jnp.float32)
            return 0
        lax.fori_loop(0, _CE, zrow, 0)
        zb = sid * _WB
        zfull, ztail = _WB // _CE, _WB % _CE
        for q in range(zfull):
            pltpu.sync_copy(rows_v[0], agg_sh.at[pl.ds(zb + q * _CE, _CE)])
        pltpu.sync_copy(rows_v[0].at[pl.ds(0, ztail)],
                        agg_sh.at[pl.ds(zb + zfull * _CE, ztail)])

        if with_cnt:
            @pl.when(cid == 0)
            def _():
                def zw(i, _):
                    wrk_v[pl.ds(i * 16, 16)] = jnp.zeros((16,), jnp.float32)
                    return 0
                lax.fori_loop(0, _CE // 16, zw, 0)
                for q in range(_WB // _CE):
                    pltpu.sync_copy(wrk_v, cnt_sh.at[pl.ds(zb + q * _CE, _CE)])
                pltpu.sync_copy(wrk_v.at[pl.ds(0, _WB % _CE)],
                                cnt_sh.at[pl.ds(zb + (_WB // _CE) * _CE,
                                                _WB % _CE)])

                def ow(i, _):
                    wrk_v[pl.ds(i * 16, 16)] = jnp.ones((16,), jnp.float32)
                    return 0
                lax.fori_loop(0, _CE // 16, ow, 0)

        plsc.subcore_barrier()

        base = sid * n_my
        nm2 = n_my // 2

        def ld_issue(i, b):
            cb = (base + i) * _CE
            pltpu.async_copy(src_h.at[pl.ds(cb, _CE)], src_v[b], sem_l[b])
            if weighted:
                pltpu.async_copy(ea_h.at[pl.ds(cb, _CE)], ea_v[b], sem_l[b])

        def ld_wait(b):
            pltpu.make_async_copy(src_h.at[pl.ds(0, _CE)], src_v[b],
                                  sem_l[b]).wait()
            if weighted:
                pltpu.make_async_copy(ea_h.at[pl.ds(0, _CE)], ea_v[b],
                                      sem_l[b]).wait()

        def gather_issue(b):
            @pl.when(cid == 0)
            def _():
                pltpu.async_copy(t_lo_h.at[src_v[b]], rows_v[b], sem_g[b])

            @pl.when(cid == 1)
            def _():
                pltpu.async_copy(t_hi_h.at[src_v[b]], rows_v[b], sem_g[b])

        def gather_wait(b):
            pltpu.make_async_copy(t_lo_h.at[src_v[b]], rows_v[b],
                                  sem_g[b]).wait()

        def scatter_issue(b):
            pltpu.async_copy(rows_v[b], agg_sh.at[dst_v[b]], sem_s[b],
                             add=True)
            if with_cnt:
                @pl.when(cid == 0)
                def _():
                    pltpu.async_copy(wrk_v, cnt_sh.at[dst_v[b]], sem_c[b],
                                     add=True)

        def scatter_wait(b):
            pltpu.make_async_copy(rows_v[b], agg_sh.at[dst_v[b]],
                                  sem_s[b]).wait()
            if with_cnt:
                @pl.when(cid == 0)
                def _():
                    pltpu.make_async_copy(wrk_v, cnt_sh.at[dst_v[b]],
                                          sem_c[b]).wait()

        ld_issue(0, 0)

        def pair(p, _):
            for b in (0, 1):
                i = p * 2 + b
                ld_wait(b)

                @pl.when(p > 0)
                def _():
                    scatter_wait(b)

                gather_issue(b)
                if b == 0:
                    ld_issue(i + 1, 1)
                else:
                    @pl.when(p < nm2 - 1)
                    def _():
                        ld_issue(i + 1, 0)
                cb = (base + i) * _CE
                pltpu.sync_copy(dst_h.at[pl.ds(cb, _CE)], dst_v[b])
                gather_wait(b)
                if weighted:
                    def scale(g, _):
                        eav = ea_v[b][pl.ds(g * 16, 16)]
                        for u in range(16):
                            e = g * 16 + u
                            rows_v[b][e, :] = rows_v[b][e, :] * eav[u]
                        return 0
                    lax.fori_loop(0, _CE // 16, scale, 0)
                scatter_issue(b)
            return 0
        lax.fori_loop(0, nm2, pair, 0)
        scatter_wait(0)
        scatter_wait(1)

        plsc.subcore_barrier()

        wb = sid * _WB

        @pl.when(jnp.logical_and(cid == 0, sid < _NSUB - 1))
        def _():
            pltpu.sync_copy(agg_sh.at[pl.ds(wb, _WB)], out_lo.at[pl.ds(wb, _WB)])

        @pl.when(jnp.logical_and(cid == 0, sid == _NSUB - 1))
        def _():
            pltpu.sync_copy(agg_sh.at[pl.ds(15 * _WB, _WB_LAST)],
                            out_lo.at[pl.ds(15 * _WB, _WB_LAST)])

        @pl.when(jnp.logical_and(cid == 1, sid < _NSUB - 1))
        def _():
            pltpu.sync_copy(agg_sh.at[pl.ds(wb, _WB)], out_hi.at[pl.ds(wb, _WB)])

        @pl.when(jnp.logical_and(cid == 1, sid == _NSUB - 1))
        def _():
            pltpu.sync_copy(agg_sh.at[pl.ds(15 * _WB, _WB_LAST)],
                            out_hi.at[pl.ds(15 * _WB, _WB_LAST)])

        if with_cnt:
            @pl.when(jnp.logical_and(cid == 0, sid < _NSUB - 1))
            def _():
                pltpu.sync_copy(cnt_sh.at[pl.ds(wb, _WB)],
                                cnt_out.at[pl.ds(wb, _WB)])

            @pl.when(jnp.logical_and(cid == 0, sid == _NSUB - 1))
            def _():
                pltpu.sync_copy(cnt_sh.at[pl.ds(15 * _WB, _WB_LAST)],
                                cnt_out.at[pl.ds(15 * _WB, _WB_LAST)])

    out_type = [jax.ShapeDtypeStruct((_NP, 16), jnp.float32),
                jax.ShapeDtypeStruct((_NP, 16), jnp.float32)]
    if with_cnt:
        out_type.append(jax.ShapeDtypeStruct((_NP,), jnp.float32))
    scratch = [pltpu.VMEM_SHARED((_ROWS, 16), jnp.float32)]
    if with_cnt:
        scratch.append(pltpu.VMEM_SHARED((_ROWS,), jnp.float32))
    scratch += [pltpu.VMEM((_CE,), jnp.int32)] * 2
    scratch += [pltpu.VMEM((_CE,), jnp.int32)] * 2
    if weighted:
        scratch += [pltpu.VMEM((_CE,), jnp.float32)] * 2
    scratch += [pltpu.VMEM((_CE, 16), jnp.float32)] * 2
    if with_cnt:
        scratch.append(pltpu.VMEM((_CE,), jnp.float32))
    scratch += [pltpu.SemaphoreType.DMA] * 6
    if with_cnt:
        scratch += [pltpu.SemaphoreType.DMA] * 2

    mesh = plsc.VectorSubcoreMesh(core_axis_name="c", subcore_axis_name="s",
                                  num_cores=_NC, num_subcores=_NSUB)
    args = [t_lo, t_hi, src, dst] + ([ea] if weighted else [])
    return pl.kernel(body, out_type=out_type, mesh=mesh,
                     compiler_params=pltpu.CompilerParams(
                         use_tc_tiling_on_sc=False),
                     scratch_types=scratch)(*args)


def _sc_belongs(t3l, t3h, t7l, t7h, src, dst):
    e_pad = src.shape[0]
    n_my = e_pad // (_CE * _NSUB)

    def body(t3l_h, t3h_h, t7l_h, t7h_h, src_h, dst_h,
             o3l, o3h, o7l, o7h, cnt_out,
             a3_sh, a7_sh, cnt_sh, src_v, dst_v, rows_v, wrk_v, sem_g):
        cid = lax.axis_index("c")
        sid = lax.axis_index("s")

        def zrow(i, _):
            rows_v[i, :] = jnp.zeros((16,), jnp.float32)
            return 0
        lax.fori_loop(0, 8, zrow, 0)

        def zw(i, _):
            wrk_v[pl.ds(i * 16, 16)] = jnp.zeros((16,), jnp.float32)
            return 0
        lax.fori_loop(0, _CE // 16, zw, 0)

        zb = sid * 8
        pltpu.sync_copy(rows_v.at[pl.ds(0, 8)], a3_sh.at[pl.ds(zb, 8)])
        pltpu.sync_copy(rows_v.at[pl.ds(0, 8)], a7_sh.at[pl.ds(zb, 8)])

        @pl.when(cid == 0)
        def _():
            pltpu.sync_copy(wrk_v.at[pl.ds(0, 8)], cnt_sh.at[pl.ds(zb, 8)])

            def ow(i, _):
                wrk_v[pl.ds(i * 16, 16)] = jnp.ones((16,), jnp.float32)
                return 0
            lax.fori_loop(0, _CE // 16, ow, 0)

        plsc.subcore_barrier()

        def chunk(j, _):
            cb = (sid * n_my + j) * _CE
            pltpu.sync_copy(src_h.at[pl.ds(cb, _CE)], src_v)
            pltpu.sync_copy(dst_h.at[pl.ds(cb, _CE)], dst_v)

            @pl.when(cid == 0)
            def _():
                pltpu.async_copy(t3l_h.at[src_v], rows_v, sem_g).wait()
                pltpu.sync_copy(rows_v, a3_sh.at[dst_v], add=True)
                pltpu.async_copy(t7l_h.at[src_v], rows_v, sem_g).wait()
                pltpu.sync_copy(rows_v, a7_sh.at[dst_v], add=True)
                pltpu.sync_copy(wrk_v, cnt_sh.at[dst_v], add=True)

            @pl.when(cid == 1)
            def _():
                pltpu.async_copy(t3h_h.at[src_v], rows_v, sem_g).wait()
                pltpu.sync_copy(rows_v, a3_sh.at[dst_v], add=True)
                pltpu.async_copy(t7h_h.at[src_v], rows_v, sem_g).wait()
                pltpu.sync_copy(rows_v, a7_sh.at[dst_v], add=True)
            return 0
        lax.fori_loop(0, n_my, chunk, 0)

        plsc.subcore_barrier()

        def zrow2(i, _):
            rows_v[i, :] = jnp.zeros((16,), jnp.float32)
            return 0
        lax.fori_loop(0, 8, zrow2, 0)
        wb = sid * 8

        @pl.when(jnp.logical_and(cid == 0, sid < 15))
        def _():
            pltpu.sync_copy(a3_sh.at[pl.ds(wb, 8)], o3l.at[pl.ds(wb, 8)])
            pltpu.sync_copy(a7_sh.at[pl.ds(wb, 8)], o7l.at[pl.ds(wb, 8)])
            pltpu.sync_copy(cnt_sh.at[pl.ds(wb, 8)], cnt_out.at[pl.ds(wb, 8)])

        @pl.when(jnp.logical_and(cid == 1, sid < 15))
        def _():
            pltpu.sync_copy(a3_sh.at[pl.ds(wb, 8)], o3h.at[pl.ds(wb, 8)])
            pltpu.sync_copy(a7_sh.at[pl.ds(wb, 8)], o7h.at[pl.ds(wb, 8)])

        @pl.when(jnp.logical_and(cid == 0, sid == 15))
        def _():
            pltpu.sync_copy(rows_v.at[pl.ds(0, 8)], o3l.at[pl.ds(120, 8)])
            pltpu.sync_copy(rows_v.at[pl.ds(0, 8)], o7l.at[pl.ds(120, 8)])

            def zw2(i, _):
                wrk_v[pl.ds(i * 16, 16)] = jnp.zeros((16,), jnp.float32)
                return 0
            lax.fori_loop(0, 1, zw2, 0)
            pltpu.sync_copy(wrk_v.at[pl.ds(0, 8)], cnt_out.at[pl.ds(120, 8)])

        @pl.when(jnp.logical_and(cid == 1, sid == 15))
        def _():
            pltpu.sync_copy(rows_v.at[pl.ds(0, 8)], o3h.at[pl.ds(120, 8)])
            pltpu.sync_copy(rows_v.at[pl.ds(0, 8)], o7h.at[pl.ds(120, 8)])

    sm = jax.ShapeDtypeStruct((128, 16), jnp.float32)
    out_type = [sm, sm, sm, sm, jax.ShapeDtypeStruct((128,), jnp.float32)]
    scratch = [pltpu.VMEM_SHARED((128, 16), jnp.float32),
               pltpu.VMEM_SHARED((128, 16), jnp.float32),
               pltpu.VMEM_SHARED((128,), jnp.float32),
               pltpu.VMEM((_CE,), jnp.int32), pltpu.VMEM((_CE,), jnp.int32),
               pltpu.VMEM((_CE, 16), jnp.float32),
               pltpu.VMEM((_CE,), jnp.float32),
               pltpu.SemaphoreType.DMA]
    mesh = plsc.VectorSubcoreMesh(core_axis_name="c", subcore_axis_name="s",
                                  num_cores=_NC, num_subcores=_NSUB)
    return pl.kernel(body, out_type=out_type, mesh=mesh,
                     compiler_params=pltpu.CompilerParams(
                         use_tc_tiling_on_sc=False),
                     scratch_types=scratch)(t3l, t3h, t7l, t7h, src, dst)


def _pad_edges(ei, ea, dump=_N, gran=2 * _CE * _NSUB):
    src, dst = ei[0], ei[1]
    e = src.shape[0]
    e_pad = -(-e // gran) * gran
    pad = e_pad - e
    src = jnp.concatenate([src, jnp.zeros((pad,), src.dtype)])
    dst = jnp.concatenate([dst, jnp.full((pad,), dump, dst.dtype)])
    if ea is not None:
        ea = jnp.concatenate([ea, jnp.zeros((pad,), ea.dtype)])
    return src, dst, ea


def kernel(x_user, x_poi, x_cate, ea_pv, ea_rev_pv, ea_trans,
           user_lin_W, user_lin_b, poi_lin_W, poi_lin_b, cate_lin_W,
           cate_lin_b, cate_emb,
           W1_u_W, W1_u_b, W1_p_W, W1_p_b, W1_c_W, W1_c_b, W1_pp_W, W1_pp_b,
           W2_u_W, W2_u_b, W2_p_W, W2_p_b, W2_c_W, W2_c_b, W2_pp_W, W2_pp_b,
           ei_pv, ei_rev_pv, ei_belongs, ei_trans):
    nc = x_cate.shape[0]

    s_rev, d_rev, ea_rev = _pad_edges(ei_rev_pv, ea_rev_pv)
    s_pv, d_pv, ea_pvp = _pad_edges(ei_pv, ea_pv)
    s_bl, d_bl, _ = _pad_edges(ei_belongs, None, dump=120, gran=_CE * _NSUB)
    s_tr, d_tr, ea_trp = _pad_edges(ei_trans, ea_trans)

    i8 = jnp.eye(8, dtype=jnp.float32)

    def bd4(w):
        wt = w.T
        return jnp.stack([jnp.kron(i8, wt[:16, :16]),
                          jnp.kron(i8, wt[16:, :16]),
                          jnp.kron(i8, wt[:16, 16:]),
                          jnp.kron(i8, wt[16:, 16:])])

    def bd2(w):
        wt = w.T
        return jnp.stack([jnp.kron(i8, wt[:, :16]), jnp.kron(i8, wt[:, 16:])])

    def pb(b):
        return jnp.stack([jnp.tile(b[:16], 8), jnp.tile(b[16:], 8)])[:, None, :]

    xu16 = jnp.concatenate([x_user.reshape(-1, 16),
                            jnp.zeros((300, 16), jnp.float32)])
    xp24 = jnp.concatenate([x_poi.reshape(-1, 24),
                            jnp.zeros((300, 24), jnp.float32)])
    xc8 = jnp.concatenate([x_cate,
                           jnp.zeros((4, 1), jnp.float32)]).reshape(-1, 8)
    cem = jnp.concatenate([cate_emb[:nc], jnp.zeros((4, 32), jnp.float32)])
    cem2 = jnp.stack([cem[:, :16].reshape(-1, 128),
                      cem[:, 16:].reshape(-1, 128)])

    (uhl, uhh, phl, phh, t1l, t1h, t3l, t3h, t7l, t7h) = _encoders(
        xu16, xp24, xc8, bd2(user_lin_W), pb(user_lin_b),
        bd2(poi_lin_W), pb(poi_lin_b), bd2(cate_lin_W), pb(cate_lin_b),
        cem2, bd4(W1_p_W), pb(W1_p_b), bd4(W1_c_W), pb(W1_c_b),
        bd4(W2_c_W), pb(W2_c_b))

    def sc(tl, th, s, d, ea, with_cnt):
        r = _sc_pass(tl.reshape(-1, 16), th.reshape(-1, 16), s, d, ea,
                     with_cnt=with_cnt)
        if with_cnt:
            return (r[0].reshape(-1, 128), r[1].reshape(-1, 128),
                    r[2].reshape(-1, 8))
        return r[0].reshape(-1, 128), r[1].reshape(-1, 128)

    a3l, a3h, a7l, a7h, cnt_bl = _sc_belongs(
        t3l.reshape(-1, 16), t3h.reshape(-1, 16),
        t7l.reshape(-1, 16), t7h.reshape(-1, 16), s_bl, d_bl)
    a3lh = (a3l.reshape(-1, 128), a3h.reshape(-1, 128))
    a7lh = (a7l.reshape(-1, 128), a7h.reshape(-1, 128))
    cnt_bl8 = cnt_bl.reshape(-1, 8)

    a1l, a1h, cnt_rev = sc(t1l, t1h, s_rev, d_rev, ea_rev, True)
    uhl1, uhh1, t2l, t2h = _epi_single((uhl, uhh), (a1l, a1h), cnt_rev,
                                       bd4(W1_u_W), pb(W1_u_b))

    a2l, a2h, cnt_pv = sc(t2l, t2h, s_pv, d_pv, ea_pvp, True)
    phl1a, phh1a, t4l, t4h = _epi_double((phl, phh), (a2l, a2h), cnt_pv,
                                         a3lh, cnt_bl8,
                                         bd4(W1_pp_W), pb(W1_pp_b))

    a4l, a4h, cnt_tr = sc(t4l, t4h, s_tr, d_tr, ea_trp, True)
    phl1, phh1, t5l, t5h = _epi_single((phl1a, phh1a), (a4l, a4h), cnt_tr,
                                       bd4(W2_p_W), pb(W2_p_b))

    a5l, a5h = sc(t5l, t5h, s_rev, d_rev, ea_rev, False)
    uhl2, uhh2, t6l, t6h = _epi_single((uhl1, uhh1), (a5l, a5h), cnt_rev,
                                       bd4(W2_u_W), pb(W2_u_b))

    a6l, a6h = sc(t6l, t6h, s_pv, d_pv, ea_pvp, False)
    phl2a, phh2a, t8l, t8h = _epi_double((phl1, phh1), (a6l, a6h), cnt_pv,
                                         a7lh, cnt_bl8,
                                         bd4(W2_pp_W), pb(W2_pp_b))

    a8l, a8h = sc(t8l, t8h, s_tr, d_tr, ea_trp, False)
    pol, poh = _epi_final((phl2a, phh2a), (a8l, a8h), cnt_tr)
    return jnp.concatenate([pol.reshape(-1, 16)[: x_poi.shape[0]],
                            poh.reshape(-1, 16)[: x_poi.shape[0]]], axis=1)


# speedup vs baseline: 1.0611x; 1.0012x over previous
import jax
import jax.numpy as jnp
from jax import lax
from jax.experimental import pallas as pl
from jax.experimental.pallas import tpu as pltpu
from jax.experimental.pallas import tpu_sc as plsc

_NB = 2000
_NC = 2
_NSUB = 16
_N = 100000
_ROWS = 100096
_CE = 512
_WB = 6256
_WB_LAST = _N - 15 * _WB


_B8 = 1600
_NP = 102400


def _cnt_expand(cnt8):
    lane = jax.lax.broadcasted_iota(jnp.int32, (8, 128), 1) // 16
    node = jax.lax.broadcasted_iota(jnp.int32, (8, 128), 0)
    k8 = (lane == node).astype(jnp.float32)
    return jnp.maximum(
        jnp.dot(cnt8, k8, preferred_element_type=jnp.float32), 1.0)


def _xform(lo, hi, bd_ref, bl_ref, bh_ref):
    t_lo = (jnp.dot(lo, bd_ref[0], preferred_element_type=jnp.float32)
            + jnp.dot(hi, bd_ref[1], preferred_element_type=jnp.float32)
            + bl_ref[...])
    t_hi = (jnp.dot(lo, bd_ref[2], preferred_element_type=jnp.float32)
            + jnp.dot(hi, bd_ref[3], preferred_element_type=jnp.float32)
            + bh_ref[...])
    return t_lo, t_hi


def _enc_body(xu_ref, xp_ref, xc_ref, bdu_ref, bul_ref, buh_ref,
              bdp_ref, bpl_ref, bph_ref, bdc_ref, bcl_ref, bch_ref,
              ceml_ref, cemh_ref, bd1p_ref, b1pl_ref, b1ph_ref,
              bd1c_ref, b1cl_ref, b1ch_ref, bd2c_ref, b2cl_ref, b2ch_ref,
              uhl_ref, uhh_ref, phl_ref, phh_ref, t1l_ref, t1h_ref,
              t3l_ref, t3h_ref, t7l_ref, t7h_ref):
    xu = xu_ref[...]
    uhl = jnp.maximum(jnp.dot(xu, bdu_ref[0],
                              preferred_element_type=jnp.float32)
                      + bul_ref[...], 0.0)
    uhh = jnp.maximum(jnp.dot(xu, bdu_ref[1],
                              preferred_element_type=jnp.float32)
                      + buh_ref[...], 0.0)
    xp = xp_ref[...]
    phl = jnp.maximum(jnp.dot(xp, bdp_ref[0],
                              preferred_element_type=jnp.float32)
                      + bpl_ref[...], 0.0)
    phh = jnp.maximum(jnp.dot(xp, bdp_ref[1],
                              preferred_element_type=jnp.float32)
                      + bph_ref[...], 0.0)
    uhl_ref[...] = uhl
    uhh_ref[...] = uhh
    phl_ref[...] = phl
    phh_ref[...] = phh
    t1l, t1h = _xform(phl, phh, bd1p_ref, b1pl_ref, b1ph_ref)
    t1l_ref[...] = t1l
    t1h_ref[...] = t1h

    @pl.when(pl.program_id(0) == 0)
    def _():
        xc = xc_ref[...]
        chl = jnp.maximum(jnp.dot(xc, bdc_ref[0],
                                  preferred_element_type=jnp.float32)
                          + bcl_ref[...], 0.0) + ceml_ref[...]
        chh = jnp.maximum(jnp.dot(xc, bdc_ref[1],
                                  preferred_element_type=jnp.float32)
                          + bch_ref[...], 0.0) + cemh_ref[...]
        t3l, t3h = _xform(chl, chh, bd1c_ref, b1cl_ref, b1ch_ref)
        t3l_ref[...] = t3l
        t3h_ref[...] = t3h
        t7l, t7h = _xform(chl, chh, bd2c_ref, b2cl_ref, b2ch_ref)
        t7l_ref[...] = t7l
        t7h_ref[...] = t7h


def _encoders(xu16, xp24, xc8, bdu, bu2, bdp, bp2, bdc, bc2, cem2,
              bd1p, b1p2, bd1c, b1c2, bd2c, b2c2):
    m = xu16.shape[0]
    mc = xc8.shape[0]
    grid = (m // _B8,)
    blk = lambda c: pl.BlockSpec((_B8, c), lambda i: (i, 0))
    full = lambda r, c: pl.BlockSpec((r, c), lambda i: (0, 0))
    pk = jax.ShapeDtypeStruct((m, 128), jnp.float32)
    pkc = jax.ShapeDtypeStruct((mc, 128), jnp.float32)
    return pl.pallas_call(
        _enc_body,
        grid=grid,
        in_specs=[blk(16), blk(24), full(mc, 8),
                  pl.BlockSpec((2, 16, 128), lambda i: (0, 0, 0)),
                  full(1, 128), full(1, 128),
                  pl.BlockSpec((2, 24, 128), lambda i: (0, 0, 0)),
                  full(1, 128), full(1, 128),
                  pl.BlockSpec((2, 8, 128), lambda i: (0, 0, 0)),
                  full(1, 128), full(1, 128),
                  full(mc, 128), full(mc, 128),
                  pl.BlockSpec((4, 128, 128), lambda i: (0, 0, 0)),
                  full(1, 128), full(1, 128),
                  pl.BlockSpec((4, 128, 128), lambda i: (0, 0, 0)),
                  full(1, 128), full(1, 128),
                  pl.BlockSpec((4, 128, 128), lambda i: (0, 0, 0)),
                  full(1, 128), full(1, 128)],
        out_specs=[blk(128), blk(128), blk(128), blk(128), blk(128),
                   blk(128), full(mc, 128), full(mc, 128), full(mc, 128),
                   full(mc, 128)],
        out_shape=[pk, pk, pk, pk, pk, pk, pkc, pkc, pkc, pkc],
    )(xu16, xp24, xc8, bdu, bu2[0], bu2[1], bdp, bp2[0], bp2[1],
      bdc, bc2[0], bc2[1], cem2[0], cem2[1], bd1p, b1p2[0], b1p2[1],
      bd1c, b1c2[0], b1c2[1], bd2c, b2c2[0], b2c2[1])


def _epi1_body(hl_ref, hh_ref, al_ref, ah_ref, cnt_ref, bd_ref, bl_ref,
               bh_ref, hlo_ref, hho_ref, tl_ref, th_ref):
    ce = _cnt_expand(cnt_ref[...])
    hl = hl_ref[...] + jnp.maximum(al_ref[...] / ce, 0.0)
    hh = hh_ref[...] + jnp.maximum(ah_ref[...] / ce, 0.0)
    hlo_ref[...] = hl
    hho_ref[...] = hh
    tl, th = _xform(hl, hh, bd_ref, bl_ref, bh_ref)
    tl_ref[...] = tl
    th_ref[...] = th


def _epi2_body(hl_ref, hh_ref, al_ref, ah_ref, cnt_a_ref,
               bl2_ref, bh2_ref, cnt_b_ref, bd_ref, bl_ref, bh_ref,
               hlo_ref, hho_ref, tl_ref, th_ref):
    ca = _cnt_expand(cnt_a_ref[...])
    cb = _cnt_expand(cnt_b_ref[...])
    g = jnp.where(pl.program_id(0) == 0, 1.0, 0.0)
    zpad = jnp.zeros((_B8 - 16, 128), jnp.float32)
    addl = jnp.concatenate([jnp.maximum(bl2_ref[...] / cb, 0.0), zpad], 0)
    addh = jnp.concatenate([jnp.maximum(bh2_ref[...] / cb, 0.0), zpad], 0)
    hl = hl_ref[...] + jnp.maximum(al_ref[...] / ca, 0.0) + g * addl
    hh = hh_ref[...] + jnp.maximum(ah_ref[...] / ca, 0.0) + g * addh
    hlo_ref[...] = hl
    hho_ref[...] = hh
    tl, th = _xform(hl, hh, bd_ref, bl_ref, bh_ref)
    tl_ref[...] = tl
    th_ref[...] = th


def _epiF_body(hl_ref, hh_ref, al_ref, ah_ref, cnt_ref, hlo_ref, hho_ref):
    ce = _cnt_expand(cnt_ref[...])
    hlo_ref[...] = hl_ref[...] + jnp.maximum(al_ref[...] / ce, 0.0)
    hho_ref[...] = hh_ref[...] + jnp.maximum(ah_ref[...] / ce, 0.0)


def _epi_single(hlh, agg_lh, cnt8, bd, b2):
    m = hlh[0].shape[0]
    grid = (m // _B8,)
    blk = lambda c: pl.BlockSpec((_B8, c), lambda i: (i, 0))
    full = lambda r, c: pl.BlockSpec((r, c), lambda i: (0, 0))
    pk = jax.ShapeDtypeStruct((m, 128), jnp.float32)
    return pl.pallas_call(
        _epi1_body, grid=grid,
        in_specs=[blk(128), blk(128), blk(128), blk(128), blk(8),
                  pl.BlockSpec((4, 128, 128), lambda i: (0, 0, 0)),
                  full(1, 128), full(1, 128)],
        out_specs=[blk(128), blk(128), blk(128), blk(128)],
        out_shape=[pk, pk, pk, pk],
    )(hlh[0], hlh[1], agg_lh[0], agg_lh[1], cnt8, bd, b2[0], b2[1])


def _epi_double(hlh, agg_a_lh, cnt_a8, agg_b_lh, cnt_b8, bd, b2):
    m = hlh[0].shape[0]
    grid = (m // _B8,)
    blk = lambda c: pl.BlockSpec((_B8, c), lambda i: (i, 0))
    full = lambda r, c: pl.BlockSpec((r, c), lambda i: (0, 0))
    pk = jax.ShapeDtypeStruct((m, 128), jnp.float32)
    return pl.pallas_call(
        _epi2_body, grid=grid,
        in_specs=[blk(128), blk(128), blk(128), blk(128), blk(8),
                  full(16, 128), full(16, 128), full(16, 8),
                  pl.BlockSpec((4, 128, 128), lambda i: (0, 0, 0)),
                  full(1, 128), full(1, 128)],
        out_specs=[blk(128), blk(128), blk(128), blk(128)],
        out_shape=[pk, pk, pk, pk],
    )(hlh[0], hlh[1], agg_a_lh[0], agg_a_lh[1], cnt_a8,
      agg_b_lh[0], agg_b_lh[1], cnt_b8, bd, b2[0], b2[1])


def _epi_final(hlh, agg_lh, cnt8):
    m = hlh[0].shape[0]
    grid = (m // _B8,)
    blk = lambda c: pl.BlockSpec((_B8, c), lambda i: (i, 0))
    pk = jax.ShapeDtypeStruct((m, 128), jnp.float32)
    return pl.pallas_call(
        _epiF_body, grid=grid,
        in_specs=[blk(128), blk(128), blk(128), blk(128), blk(8)],
        out_specs=[blk(128), blk(128)],
        out_shape=[pk, pk],
    )(hlh[0], hlh[1], agg_lh[0], agg_lh[1], cnt8)


def _sc_pass(t_lo, t_hi, src, dst, ea, *, with_cnt):
    weighted = ea is not None
    e_pad = src.shape[0]
    n_my = e_pad // (_CE * _NSUB)

    def body(*refs):
        iota16 = lax.iota(jnp.int32, 16)
        it = iter(refs)
        t_lo_h = next(it)
        t_hi_h = next(it)
        src_h = next(it)
        dst_h = next(it)
        ea_h = next(it) if weighted else None
        out_lo = next(it)
        out_hi = next(it)
        cnt_out = next(it) if with_cnt else None
        agg_sh = next(it)
        cnt_sh = next(it) if with_cnt else None
        src_v = (next(it), next(it))
        dst_v = (next(it), next(it))
        ea_v = (next(it), next(it)) if weighted else None
        rows_v = (next(it), next(it))
        wrk_v = next(it) if with_cnt else None
        sem_l = (next(it), next(it))
        sem_g = (next(it), next(it))
        sem_s = (next(it), next(it))
        sem_c = (next(it), next(it)) if with_cnt else None

        cid = lax.axis_index("c")
        sid = lax.axis_index("s")

        def zrow(i, _):
            rows_v[0][i, :] = jnp.zeros((16,), jnp.float32)
            return 0
        lax.fori_loop(0, _CE, zrow, 0)
        zb = sid * _WB
        zfull, ztail = _WB // _CE, _WB % _CE
        for q in range(zfull):
            pltpu.sync_copy(rows_v[0], agg_sh.at[pl.ds(zb + q * _CE, _CE)])
        pltpu.sync_copy(rows_v[0].at[pl.ds(0, ztail)],
                        agg_sh.at[pl.ds(zb + zfull * _CE, ztail)])

        if with_cnt:
            @pl.when(cid == 0)
            def _():
                def zw(i, _):
                    wrk_v[pl.ds(i * 16, 16)] = jnp.zeros((16,), jnp.float32)
                    return 0
                lax.fori_loop(0, _CE // 16, zw, 0)
                for q in range(_WB // _CE):
                    pltpu.sync_copy(wrk_v, cnt_sh.at[pl.ds(zb + q * _CE, _CE)])
                pltpu.sync_copy(wrk_v.at[pl.ds(0, _WB % _CE)],
                                cnt_sh.at[pl.ds(zb + (_WB // _CE) * _CE,
                                                _WB % _CE)])

                def ow(i, _):
                    wrk_v[pl.ds(i * 16, 16)] = jnp.ones((16,), jnp.float32)
                    return 0
                lax.fori_loop(0, _CE // 16, ow, 0)

        plsc.subcore_barrier()

        base = sid * n_my
        nm2 = n_my // 2

        def ld_issue(i, b):
            cb = (base + i) * _CE
            pltpu.async_copy(src_h.at[pl.ds(cb, _CE)], src_v[b], sem_l[b])
            if weighted:
                pltpu.async_copy(ea_h.at[pl.ds(cb, _CE)], ea_v[b], sem_l[b])

        def ld_wait(b):
            pltpu.make_async_copy(src_h.at[pl.ds(0, _CE)], src_v[b],
                                  sem_l[b]).wait()
            if weighted:
                pltpu.make_async_copy(ea_h.at[pl.ds(0, _CE)], ea_v[b],
                                      sem_l[b]).wait()

        def gather_issue(b):
            @pl.when(cid == 0)
            def _():
                pltpu.async_copy(t_lo_h.at[src_v[b]], rows_v[b], sem_g[b])

            @pl.when(cid == 1)
            def _():
                pltpu.async_copy(t_hi_h.at[src_v[b]], rows_v[b], sem_g[b])

        def gather_wait(b):
            pltpu.make_async_copy(t_lo_h.at[src_v[b]], rows_v[b],
                                  sem_g[b]).wait()

        def scatter_issue(b):
            pltpu.async_copy(rows_v[b], agg_sh.at[dst_v[b]], sem_s[b],
                             add=True)
            if with_cnt:
                @pl.when(cid == 0)
                def _():
                    pltpu.async_copy(wrk_v, cnt_sh.at[dst_v[b]], sem_c[b],
                                     add=True)

        def scatter_wait(b):
            pltpu.make_async_copy(rows_v[b], agg_sh.at[dst_v[b]],
                                  sem_s[b]).wait()
            if with_cnt:
                @pl.when(cid == 0)
                def _():
                    pltpu.make_async_copy(wrk_v, cnt_sh.at[dst_v[b]],
                                          sem_c[b]).wait()

        ld_issue(0, 0)

        def pair(p, _):
            for b in (0, 1):
                i = p * 2 + b
                ld_wait(b)

                @pl.when(p > 0)
                def _():
                    scatter_wait(b)

                gather_issue(b)
                if b == 0:
                    ld_issue(i + 1, 1)
                else:
                    @pl.when(p < nm2 - 1)
                    def _():
                        ld_issue(i + 1, 0)
                cb = (base + i) * _CE
                pltpu.sync_copy(dst_h.at[pl.ds(cb, _CE)], dst_v[b])
                gather_wait(b)
                if weighted:
                    def scale(g, _):
                        eav = ea_v[b][pl.ds(g * 16, 16)]
                        for u in range(16):
                            e = g * 16 + u
                            rows_v[b][e, :] = rows_v[b][e, :] * eav[u]
                        return 0
                    lax.fori_loop(0, _CE // 16, scale, 0)
                scatter_issue(b)
            return 0
        lax.fori_loop(0, nm2, pair, 0)
        scatter_wait(0)
        scatter_wait(1)

        plsc.subcore_barrier()

        wb = sid * _WB

        @pl.when(jnp.logical_and(cid == 0, sid < _NSUB - 1))
        def _():
            pltpu.sync_copy(agg_sh.at[pl.ds(wb, _WB)], out_lo.at[pl.ds(wb, _WB)])

        @pl.when(jnp.logical_and(cid == 0, sid == _NSUB - 1))
        def _():
            pltpu.sync_copy(agg_sh.at[pl.ds(15 * _WB, _WB_LAST)],
                            out_lo.at[pl.ds(15 * _WB, _WB_LAST)])

        @pl.when(jnp.logical_and(cid == 1, sid < _NSUB - 1))
        def _():
            pltpu.sync_copy(agg_sh.at[pl.ds(wb, _WB)], out_hi.at[pl.ds(wb, _WB)])

        @pl.when(jnp.logical_and(cid == 1, sid == _NSUB - 1))
        def _():
            pltpu.sync_copy(agg_sh.at[pl.ds(15 * _WB, _WB_LAST)],
                            out_hi.at[pl.ds(15 * _WB, _WB_LAST)])

        if with_cnt:
            @pl.when(jnp.logical_and(cid == 0, sid < _NSUB - 1))
            def _():
                pltpu.sync_copy(cnt_sh.at[pl.ds(wb, _WB)],
                                cnt_out.at[pl.ds(wb, _WB)])

            @pl.when(jnp.logical_and(cid == 0, sid == _NSUB - 1))
            def _():
                pltpu.sync_copy(cnt_sh.at[pl.ds(15 * _WB, _WB_LAST)],
                                cnt_out.at[pl.ds(15 * _WB, _WB_LAST)])

    out_type = [jax.ShapeDtypeStruct((_NP, 16), jnp.float32),
                jax.ShapeDtypeStruct((_NP, 16), jnp.float32)]
    if with_cnt:
        out_type.append(jax.ShapeDtypeStruct((_NP,), jnp.float32))
    scratch = [pltpu.VMEM_SHARED((_ROWS, 16), jnp.float32)]
    if with_cnt:
        scratch.append(pltpu.VMEM_SHARED((_ROWS,), jnp.float32))
    scratch += [pltpu.VMEM((_CE,), jnp.int32)] * 2
    scratch += [pltpu.VMEM((_CE,), jnp.int32)] * 2
    if weighted:
        scratch += [pltpu.VMEM((_CE,), jnp.float32)] * 2
    scratch += [pltpu.VMEM((_CE, 16), jnp.float32)] * 2
    if with_cnt:
        scratch.append(pltpu.VMEM((_CE,), jnp.float32))
    scratch += [pltpu.SemaphoreType.DMA] * 6
    if with_cnt:
        scratch += [pltpu.SemaphoreType.DMA] * 2

    mesh = plsc.VectorSubcoreMesh(core_axis_name="c", subcore_axis_name="s",
                                  num_cores=_NC, num_subcores=_NSUB)
    args = [t_lo, t_hi, src, dst] + ([ea] if weighted else [])
    return pl.kernel(body, out_type=out_type, mesh=mesh,
                     compiler_params=pltpu.CompilerParams(
                         use_tc_tiling_on_sc=False),
                     scratch_types=scratch)(*args)


def _sc_belongs(t3l, t3h, t7l, t7h, src, dst):
    e_pad = src.shape[0]
    n_my = e_pad // (_CE * _NSUB)

    def body(t3l_h, t3h_h, t7l_h, t7h_h, src_h, dst_h,
             o3l, o3h, o7l, o7h, cnt_out,
             a3_sh, a7_sh, cnt_sh, src_v, dst_v, rows_v, wrk_v, sem_g):
        cid = lax.axis_index("c")
        sid = lax.axis_index("s")

        def zrow(i, _):
            rows_v[i, :] = jnp.zeros((16,), jnp.float32)
            return 0
        lax.fori_loop(0, 8, zrow, 0)

        def zw(i, _):
            wrk_v[pl.ds(i * 16, 16)] = jnp.zeros((16,), jnp.float32)
            return 0
        lax.fori_loop(0, _CE // 16, zw, 0)

        zb = sid * 8
        pltpu.sync_copy(rows_v.at[pl.ds(0, 8)], a3_sh.at[pl.ds(zb, 8)])
        pltpu.sync_copy(rows_v.at[pl.ds(0, 8)], a7_sh.at[pl.ds(zb, 8)])

        @pl.when(cid == 0)
        def _():
            pltpu.sync_copy(wrk_v.at[pl.ds(0, 8)], cnt_sh.at[pl.ds(zb, 8)])

            def ow(i, _):
                wrk_v[pl.ds(i * 16, 16)] = jnp.ones((16,), jnp.float32)
                return 0
            lax.fori_loop(0, _CE // 16, ow, 0)

        plsc.subcore_barrier()

        def chunk(j, _):
            cb = (sid * n_my + j) * _CE
            pltpu.sync_copy(src_h.at[pl.ds(cb, _CE)], src_v)
            pltpu.sync_copy(dst_h.at[pl.ds(cb, _CE)], dst_v)

            @pl.when(cid == 0)
            def _():
                pltpu.async_copy(t3l_h.at[src_v], rows_v, sem_g).wait()
                pltpu.sync_copy(rows_v, a3_sh.at[dst_v], add=True)
                pltpu.async_copy(t7l_h.at[src_v], rows_v, sem_g).wait()
                pltpu.sync_copy(rows_v, a7_sh.at[dst_v], add=True)
                pltpu.sync_copy(wrk_v, cnt_sh.at[dst_v], add=True)

            @pl.when(cid == 1)
            def _():
                pltpu.async_copy(t3h_h.at[src_v], rows_v, sem_g).wait()
                pltpu.sync_copy(rows_v, a3_sh.at[dst_v], add=True)
                pltpu.async_copy(t7h_h.at[src_v], rows_v, sem_g).wait()
                pltpu.sync_copy(rows_v, a7_sh.at[dst_v], add=True)
            return 0
        lax.fori_loop(0, n_my, chunk, 0)

        plsc.subcore_barrier()

        def zrow2(i, _):
            rows_v[i, :] = jnp.zeros((16,), jnp.float32)
            return 0
        lax.fori_loop(0, 8, zrow2, 0)
        wb = sid * 8

        @pl.when(jnp.logical_and(cid == 0, sid < 15))
        def _():
            pltpu.sync_copy(a3_sh.at[pl.ds(wb, 8)], o3l.at[pl.ds(wb, 8)])
            pltpu.sync_copy(a7_sh.at[pl.ds(wb, 8)], o7l.at[pl.ds(wb, 8)])
            pltpu.sync_copy(cnt_sh.at[pl.ds(wb, 8)], cnt_out.at[pl.ds(wb, 8)])

        @pl.when(jnp.logical_and(cid == 1, sid < 15))
        def _():
            pltpu.sync_copy(a3_sh.at[pl.ds(wb, 8)], o3h.at[pl.ds(wb, 8)])
            pltpu.sync_copy(a7_sh.at[pl.ds(wb, 8)], o7h.at[pl.ds(wb, 8)])

        @pl.when(jnp.logical_and(cid == 0, sid == 15))
        def _():
            pltpu.sync_copy(rows_v.at[pl.ds(0, 8)], o3l.at[pl.ds(120, 8)])
            pltpu.sync_copy(rows_v.at[pl.ds(0, 8)], o7l.at[pl.ds(120, 8)])

            def zw2(i, _):
                wrk_v[pl.ds(i * 16, 16)] = jnp.zeros((16,), jnp.float32)
                return 0
            lax.fori_loop(0, 1, zw2, 0)
            pltpu.sync_copy(wrk_v.at[pl.ds(0, 8)], cnt_out.at[pl.ds(120, 8)])

        @pl.when(jnp.logical_and(cid == 1, sid == 15))
        def _():
            pltpu.sync_copy(rows_v.at[pl.ds(0, 8)], o3h.at[pl.ds(120, 8)])
            pltpu.sync_copy(rows_v.at[pl.ds(0, 8)], o7h.at[pl.ds(120, 8)])

    sm = jax.ShapeDtypeStruct((128, 16), jnp.float32)
    out_type = [sm, sm, sm, sm, jax.ShapeDtypeStruct((128,), jnp.float32)]
    scratch = [pltpu.VMEM_SHARED((128, 16), jnp.float32),
               pltpu.VMEM_SHARED((128, 16), jnp.float32),
               pltpu.VMEM_SHARED((128,), jnp.float32),
               pltpu.VMEM((_CE,), jnp.int32), pltpu.VMEM((_CE,), jnp.int32),
               pltpu.VMEM((_CE, 16), jnp.float32),
               pltpu.VMEM((_CE,), jnp.float32),
               pltpu.SemaphoreType.DMA]
    mesh = plsc.VectorSubcoreMesh(core_axis_name="c", subcore_axis_name="s",
                                  num_cores=_NC, num_subcores=_NSUB)
    return pl.kernel(body, out_type=out_type, mesh=mesh,
                     compiler_params=pltpu.CompilerParams(
                         use_tc_tiling_on_sc=False),
                     scratch_types=scratch)(t3l, t3h, t7l, t7h, src, dst)


def _pad_edges(ei, ea, dump=_N, gran=2 * _CE * _NSUB):
    src, dst = ei[0], ei[1]
    e = src.shape[0]
    e_pad = -(-e // gran) * gran
    pad = e_pad - e
    src = jnp.concatenate([src, jnp.zeros((pad,), src.dtype)])
    dst = jnp.concatenate([dst, jnp.full((pad,), dump, dst.dtype)])
    if ea is not None:
        ea = jnp.concatenate([ea, jnp.zeros((pad,), ea.dtype)])
    return src, dst, ea


def kernel(x_user, x_poi, x_cate, ea_pv, ea_rev_pv, ea_trans,
           user_lin_W, user_lin_b, poi_lin_W, poi_lin_b, cate_lin_W,
           cate_lin_b, cate_emb,
           W1_u_W, W1_u_b, W1_p_W, W1_p_b, W1_c_W, W1_c_b, W1_pp_W, W1_pp_b,
           W2_u_W, W2_u_b, W2_p_W, W2_p_b, W2_c_W, W2_c_b, W2_pp_W, W2_pp_b,
           ei_pv, ei_rev_pv, ei_belongs, ei_trans):
    nc = x_cate.shape[0]

    s_rev, d_rev, ea_rev = _pad_edges(ei_rev_pv, ea_rev_pv)
    s_pv, d_pv, ea_pvp = _pad_edges(ei_pv, ea_pv)
    s_bl, d_bl, _ = _pad_edges(ei_belongs, None, dump=120, gran=_CE * _NSUB)
    s_tr, d_tr, ea_trp = _pad_edges(ei_trans, ea_trans)

    i8 = jnp.eye(8, dtype=jnp.float32)

    def bd4(w):
        wt = w.T
        return jnp.stack([jnp.kron(i8, wt[:16, :16]),
                          jnp.kron(i8, wt[16:, :16]),
                          jnp.kron(i8, wt[:16, 16:]),
                          jnp.kron(i8, wt[16:, 16:])])

    def bd2(w):
        wt = w.T
        return jnp.stack([jnp.kron(i8, wt[:, :16]), jnp.kron(i8, wt[:, 16:])])

    def pb(b):
        return jnp.stack([jnp.tile(b[:16], 8), jnp.tile(b[16:], 8)])[:, None, :]

    xu16 = jnp.concatenate([x_user.reshape(-1, 16),
                            jnp.zeros((300, 16), jnp.float32)])
    xp24 = jnp.concatenate([x_poi.reshape(-1, 24),
                            jnp.zeros((300, 24), jnp.float32)])
    xc8 = jnp.concatenate([x_cate,
                           jnp.zeros((4, 1), jnp.float32)]).reshape(-1, 8)
    cem = jnp.concatenate([cate_emb[:nc], jnp.zeros((4, 32), jnp.float32)])
    cem2 = jnp.stack([cem[:, :16].reshape(-1, 128),
                      cem[:, 16:].reshape(-1, 128)])

    (uhl, uhh, phl, phh, t1l, t1h, t3l, t3h, t7l, t7h) = _encoders(
        xu16, xp24, xc8, bd2(user_lin_W), pb(user_lin_b),
        bd2(poi_lin_W), pb(poi_lin_b), bd2(cate_lin_W), pb(cate_lin_b),
        cem2, bd4(W1_p_W), pb(W1_p_b), bd4(W1_c_W), pb(W1_c_b),
        bd4(W2_c_W), pb(W2_c_b))

    def sc(tl, th, s, d, ea, with_cnt):
        r = _sc_pass(tl.reshape(-1, 16), th.reshape(-1, 16), s, d, ea,
                     with_cnt=with_cnt)
        if with_cnt:
            return (r[0].reshape(-1, 128), r[1].reshape(-1, 128),
                    r[2].reshape(-1, 8))
        return r[0].reshape(-1, 128), r[1].reshape(-1, 128)

    a3l, a3h, a7l, a7h, cnt_bl = _sc_belongs(
        t3l.reshape(-1, 16), t3h.reshape(-1, 16),
        t7l.reshape(-1, 16), t7h.reshape(-1, 16), s_bl, d_bl)
    a3lh = (a3l.reshape(-1, 128), a3h.reshape(-1, 128))
    a7lh = (a7l.reshape(-1, 128), a7h.reshape(-1, 128))
    cnt_bl8 = cnt_bl.reshape(-1, 8)

    a1l, a1h, cnt_rev = sc(t1l, t1h, s_rev, d_rev, ea_rev, True)
    uhl1, uhh1, t2l, t2h = _epi_single((uhl, uhh), (a1l, a1h), cnt_rev,
                                       bd4(W1_u_W), pb(W1_u_b))

    a2l, a2h, cnt_pv = sc(t2l, t2h, s_pv, d_pv, ea_pvp, True)
    phl1a, phh1a, t4l, t4h = _epi_double((phl, phh), (a2l, a2h), cnt_pv,
                                         a3lh, cnt_bl8,
                                         bd4(W1_pp_W), pb(W1_pp_b))

    a4l, a4h, cnt_tr = sc(t4l, t4h, s_tr, d_tr, ea_trp, True)
    phl1, phh1, t5l, t5h = _epi_single((phl1a, phh1a), (a4l, a4h), cnt_tr,
                                       bd4(W2_p_W), pb(W2_p_b))

    a5l, a5h = sc(t5l, t5h, s_rev, d_rev, ea_rev, False)
    uhl2, uhh2, t6l, t6h = _epi_single((uhl1, uhh1), (a5l, a5h), cnt_rev,
                                       bd4(W2_u_W), pb(W2_u_b))

    a6l, a6h = sc(t6l, t6h, s_pv, d_pv, ea_pvp, False)
    phl2a, phh2a, t8l, t8h = _epi_double((phl1, phh1), (a6l, a6h), cnt_pv,
                                         a7lh, cnt_bl8,
                                         bd4(W2_pp_W), pb(W2_pp_b))

    a8l, a8h = sc(t8l, t8h, s_tr, d_tr, ea_trp, False)
    pol, poh = _epi_final((phl2a, phh2a), (a8l, a8h), cnt_tr)
    return jnp.concatenate([pol.reshape(-1, 16)[: x_poi.shape[0]],
                            poh.reshape(-1, 16)[: x_poi.shape[0]]], axis=1)
